# Initial kernel scaffold; baseline (speedup 1.0000x reference)
import jax
import jax.numpy as jnp
from jax.experimental import pallas as pl


def kernel(user_embed, book_embed, user_ids, pos_book_ids, neg_book_ids):
    raise NotImplementedError("write your pallas kernel here")


# speedup vs baseline: 3.0764x; 3.0764x over previous
import functools

import jax
import jax.numpy as jnp
from jax import lax
from jax.experimental import pallas as pl
from jax.experimental.pallas import tpu as pltpu
from jax.experimental.pallas import tpu_sc as plsc

_LANES = 16
_IDXW = 128


def _sc_scores(user_embed, book_embed, uids, pids, nids, B, K, D):
    info = plsc.get_sparse_core_info()
    NC, NS = info.num_cores, info.num_subcores
    NW = NC * NS
    chunk = B // NW
    n_grp = chunk // _LANES
    n_iu = chunk // _IDXW
    n_in = chunk * K // _IDXW

    mesh = plsc.VectorSubcoreMesh(core_axis_name="c", subcore_axis_name="s")

    @functools.partial(
        pl.kernel,
        out_type=[
            jax.ShapeDtypeStruct((B,), jnp.float32),
            jax.ShapeDtypeStruct((B * K,), jnp.float32),
        ],
        mesh=mesh,
        compiler_params=pltpu.CompilerParams(
            needs_layout_passes=False,
            use_tc_tiling_on_sc=False,
        ),
        scratch_types=[
            pltpu.VMEM((n_iu, _IDXW), jnp.int32),
            pltpu.VMEM((n_iu, _IDXW), jnp.int32),
            pltpu.VMEM((n_in, _IDXW), jnp.int32),
            pltpu.VMEM((chunk, D), jnp.float32),
            pltpu.VMEM((chunk, D), jnp.float32),
            pltpu.VMEM((chunk * K, D), jnp.float32),
            pltpu.VMEM((chunk,), jnp.float32),
            pltpu.VMEM((chunk * K,), jnp.float32),
            pltpu.SemaphoreType.DMA,
        ],
    )
    def sc_kernel(uids_h, pids_h, nids_h, uemb_h, bemb_h, pos_o, neg_o,
                  idx_u, idx_p, idx_n, rows_u, rows_p, rows_n, pos_v, neg_v,
                  sem):
        wid = lax.axis_index("s") * NC + lax.axis_index("c")
        pltpu.sync_copy(uids_h.at[wid], idx_u)
        pltpu.sync_copy(pids_h.at[wid], idx_p)
        pltpu.sync_copy(nids_h.at[wid], idx_n)
        cps = []
        for j in range(n_iu):
            cps.append(pltpu.async_copy(
                uemb_h.at[idx_u.at[j]], rows_u.at[pl.ds(j * _IDXW, _IDXW)], sem))
            cps.append(pltpu.async_copy(
                bemb_h.at[idx_p.at[j]], rows_p.at[pl.ds(j * _IDXW, _IDXW)], sem))
        for j in range(n_in):
            cps.append(pltpu.async_copy(
                bemb_h.at[idx_n.at[j]], rows_n.at[pl.ds(j * _IDXW, _IDXW)], sem))
        for c in cps:
            c.wait()

        def group(g, carry):
            bvec = g * _LANES + lax.iota(jnp.int32, _LANES)
            bvecK = bvec * K
            accp = jnp.zeros((_LANES,), jnp.float32)
            accn = [jnp.zeros((_LANES,), jnp.float32) for _ in range(K)]
            for d in range(D):
                col = jnp.full((_LANES,), d, jnp.int32)
                uv = plsc.load_gather(rows_u, [bvec, col])
                pv = plsc.load_gather(rows_p, [bvec, col])
                accp = accp + uv * pv
                for k in range(K):
                    nv = plsc.load_gather(rows_n, [bvecK + k, col])
                    accn[k] = accn[k] + uv * nv
            plsc.store_scatter(pos_v, [bvec], accp)
            for k in range(K):
                plsc.store_scatter(neg_v, [bvecK + k], accn[k])
            return carry

        lax.fori_loop(0, n_grp, group, 0)
        pltpu.sync_copy(pos_v, pos_o.at[pl.ds(wid * chunk, chunk)])
        pltpu.sync_copy(neg_v, neg_o.at[pl.ds(wid * chunk * K, chunk * K)])

    return sc_kernel(uids, pids, nids, user_embed, book_embed)


def _loss_tc(pos_s, neg_s, B):
    pos2 = pos_s.reshape(-1, 128)
    neg2 = neg_s.reshape(-1, 128)

    def body(p_ref, n_ref, o_ref):
        p = p_ref[...]
        n = n_ref[...]
        lp = jnp.log(1.0 / (1.0 + jnp.exp(-p)) + 1e-10)
        ln = jnp.log(1.0 / (1.0 + jnp.exp(n)) + 1e-10)
        o_ref[0, 0] = -(jnp.sum(lp) + jnp.sum(ln)) * (1.0 / B)

    out = pl.pallas_call(
        body,
        out_shape=jax.ShapeDtypeStruct((1, 1), jnp.float32),
        out_specs=pl.BlockSpec(memory_space=pltpu.SMEM),
    )(pos2, neg2)
    return out[0, 0]


def kernel(user_embed, book_embed, user_ids, pos_book_ids, neg_book_ids):
    B = user_ids.shape[0]
    K = neg_book_ids.shape[1]
    D = user_embed.shape[1]
    info = plsc.get_sparse_core_info()
    NW = info.num_cores * info.num_subcores
    uids = user_ids.astype(jnp.int32).reshape(NW, -1, _IDXW)
    pids = pos_book_ids.astype(jnp.int32).reshape(NW, -1, _IDXW)
    nids = neg_book_ids.astype(jnp.int32).reshape(NW, -1, _IDXW)
    pos_s, neg_s = _sc_scores(user_embed, book_embed, uids, pids, nids, B, K, D)
    return _loss_tc(pos_s, neg_s, B)
